# baseline (speedup 1.0000x reference)
import functools

import jax
import jax.numpy as jnp
from jax import lax
from jax.experimental import pallas as pl
from jax.experimental.pallas import tpu as pltpu
from jax.experimental.pallas import tpu_sc as plsc

N_NODES = 10000
N_EDGES = 320000
D = 128
NUM_GRAPHS = 64
D_OUT = 40

NC = 2
NS = 16
NW = NC * NS
C = 128
CHUNKS = 80
E_W = C * CHUNKS
E_PAD = NW * E_W
N_PAD = 10240
ROWS_W = N_PAD // NS
NBUF = 2
SEG = 8
CH0 = 152
CH1 = 8

_HIGH = jax.lax.Precision.HIGHEST


def _make_sc_agg(with_deg: bool):
    mesh = plsc.VectorSubcoreMesh(core_axis_name="c", subcore_axis_name="s",
                                  num_cores=NC)
    out_type = [jax.ShapeDtypeStruct((NC * N_PAD, D), jnp.float32)]
    if with_deg:
        out_type.append(jax.ShapeDtypeStruct((NW, N_PAD), jnp.float32))
    scratch = [
        pltpu.VMEM((2, SEG, C), jnp.int32),
        pltpu.VMEM((2, SEG, C), jnp.int32),
        pltpu.VMEM((NBUF, C, D), jnp.float32),
        pltpu.VMEM_SHARED((N_PAD, D), jnp.float32),
    ]
    scratch += [pltpu.SemaphoreType.DMA] * (2 * NBUF + 2)
    if with_deg:
        scratch.append(pltpu.VMEM((N_PAD,), jnp.float32))

    out_type_arg = tuple(out_type) if with_deg else out_type[0]

    @functools.partial(
        pl.kernel, mesh=mesh, out_type=out_type_arg,
        scratch_types=tuple(scratch),
        compiler_params=pltpu.CompilerParams(needs_layout_passes=False))
    def sc_agg(feat, srcm, dstm, *refs):
        if with_deg:
            (agg_out, deg_out, src_v, dst_v, rows_v, acc,
             *sems, deg_v) = refs
        else:
            agg_out, src_v, dst_v, rows_v, acc, *sems = refs
        gsem, ssem = sems[:NBUF], sems[NBUF:2 * NBUF]
        isem = sems[2 * NBUF:2 * NBUF + 2]
        ci = lax.axis_index("c")
        si = lax.axis_index("s")
        w = ci * NS + si

        zeros16 = jnp.zeros((16,), jnp.float32)

        def _zrow(i, carry):
            for k in range(D // 16):
                rows_v[0, i, pl.ds(k * 16, 16)] = zeros16
            return carry
        lax.fori_loop(0, C, _zrow, 0)

        if with_deg:
            def _zdeg(i, carry):
                deg_v[pl.ds(i * 16, 16)] = zeros16
                return carry
            lax.fori_loop(0, N_PAD // 16, _zdeg, 0)

        base = si * ROWS_W
        for t in range(ROWS_W // C):
            pltpu.sync_copy(rows_v.at[0], acc.at[pl.ds(base + t * C, C)])
        plsc.subcore_barrier()

        ones16 = jnp.ones((16,), jnp.float32)

        chunk_base = jnp.where(ci == 0, si * CH0, NS * CH0 + si * CH1)
        nseg_w = jnp.where(ci == 0, CH0 // SEG, CH1 // SEG)

        def _load_idx(s, p):
            seg_base = chunk_base + s * SEG
            pltpu.async_copy(srcm.at[pl.ds(seg_base, SEG)], src_v.at[p],
                             isem[0])
            pltpu.async_copy(dstm.at[pl.ds(seg_base, SEG)], dst_v.at[p],
                             isem[1])

        def _wait_idx(s, p):
            seg_base = chunk_base + s * SEG
            pltpu.make_async_copy(srcm.at[pl.ds(seg_base, SEG)], src_v.at[p],
                                  isem[0]).wait()
            pltpu.make_async_copy(dstm.at[pl.ds(seg_base, SEG)], dst_v.at[p],
                                  isem[1]).wait()

        @pl.when(0 < nseg_w)
        def _prime_idx():
            _load_idx(0, 0)

        def _segment(s, carry):
            p = lax.rem(s, 2)
            np_ = lax.rem(s + 1, 2)
            _wait_idx(s, p)

            @pl.when(s + 1 < nseg_w)
            def _prefetch_idx():
                _load_idx(s + 1, np_)

            pltpu.async_copy(feat.at[src_v.at[p, 0]], rows_v.at[0], gsem[0])
            for jj in range(SEG):
                b = jj % NBUF
                nb = (jj + 1) % NBUF
                pltpu.make_async_copy(
                    feat.at[src_v.at[p, jj]], rows_v.at[b], gsem[b]).wait()
                if jj + 1 < SEG:
                    if jj >= 1:
                        pltpu.make_async_copy(
                            rows_v.at[nb], acc.at[dst_v.at[p, jj]],
                            ssem[nb]).wait()
                    pltpu.async_copy(
                        feat.at[src_v.at[p, jj + 1]], rows_v.at[nb], gsem[nb])
                pltpu.async_copy(
                    rows_v.at[b], acc.at[dst_v.at[p, jj]], ssem[b], add=True)
                if with_deg:
                    for k in range(C // 16):
                        idx = dst_v[p, jj, pl.ds(k * 16, 16)]
                        plsc.addupdate_scatter(deg_v, [idx], ones16)
            for b in range(NBUF):
                pltpu.make_async_copy(
                    rows_v.at[b], acc.at[dst_v.at[p, SEG - 1]], ssem[b]).wait()
            return carry
        lax.fori_loop(0, nseg_w, _segment, 0)

        plsc.subcore_barrier()

        out_base = ci * N_PAD + base
        for t in range(ROWS_W // C):
            pltpu.sync_copy(acc.at[pl.ds(base + t * C, C)], rows_v.at[0])
            pltpu.sync_copy(rows_v.at[0], agg_out.at[pl.ds(out_base + t * C, C)])
        if with_deg:
            pltpu.sync_copy(deg_v, deg_out.at[w])

    return sc_agg


_make_sc_agg = functools.lru_cache(maxsize=None)(_make_sc_agg)


def _layer_body(agg_ref, deg_ref, x_ref, wl_ref, wr_ref, b_ref, h_ref):
    a = agg_ref[...]
    s = a[:N_PAD] + a[N_PAD:]
    deg = jnp.sum(deg_ref[...], axis=0)
    inv = 1.0 / jnp.maximum(deg, 1.0)
    m = s * inv[:, None]
    h = (jnp.dot(m, wl_ref[...], precision=_HIGH,
                 preferred_element_type=jnp.float32)
         + jnp.dot(x_ref[...], wr_ref[...], precision=_HIGH,
                   preferred_element_type=jnp.float32)
         + b_ref[...][None, :])
    h_ref[...] = jnp.maximum(h, 0.0)


_layer_tc = pl.pallas_call(
    _layer_body,
    out_shape=jax.ShapeDtypeStruct((N_PAD, D), jnp.float32),
)


def _head_body(agg_ref, deg_ref, h1_ref, wl_ref, wr_ref, b_ref,
               bat_ref, mw1_ref, mb1_ref, mw2_ref, mb2_ref,
               cw_ref, cb_ref, out_ref):
    a = agg_ref[...]
    s2 = a[:N_PAD] + a[N_PAD:]
    deg = jnp.sum(deg_ref[...], axis=0)
    inv = 1.0 / jnp.maximum(deg, 1.0)
    m = s2 * inv[:, None]
    h = jnp.maximum(
        jnp.dot(m, wl_ref[...], precision=_HIGH,
                preferred_element_type=jnp.float32)
        + jnp.dot(h1_ref[...], wr_ref[...], precision=_HIGH,
                  preferred_element_type=jnp.float32)
        + b_ref[...][None, :], 0.0)
    bat = bat_ref[...]
    ids = lax.broadcasted_iota(jnp.int32, (NUM_GRAPHS, N_PAD), 0)
    onehot = (ids == bat[None, :]).astype(jnp.float32)
    pooled = jnp.dot(onehot, h, precision=_HIGH,
                     preferred_element_type=jnp.float32)
    cnt = jnp.sum(onehot, axis=1)
    g = pooled / jnp.maximum(cnt, 1.0)[:, None]
    g = jnp.maximum(jnp.dot(g, mw1_ref[...], precision=_HIGH,
                            preferred_element_type=jnp.float32)
                    + mb1_ref[...][None, :], 0.0)
    g = jnp.maximum(jnp.dot(g, mw2_ref[...], precision=_HIGH,
                            preferred_element_type=jnp.float32)
                    + mb2_ref[...][None, :], 0.0)
    out_ref[...] = (jnp.dot(g, cw_ref[...], precision=_HIGH,
                            preferred_element_type=jnp.float32)
                    + cb_ref[...][None, :])


_head_tc = pl.pallas_call(
    _head_body,
    out_shape=jax.ShapeDtypeStruct((NUM_GRAPHS, D_OUT), jnp.float32),
)


def kernel(x, edge_index, batch, W_l0, W_r0, b0, W_l1, W_r1, b1,
           mlp_W1, mlp_b1, mlp_W2, mlp_b2, cls_W, cls_b):
    src = edge_index[0].astype(jnp.int32)
    dst = edge_index[1].astype(jnp.int32)
    pad = E_PAD - N_EDGES
    srcm = jnp.concatenate([src, jnp.zeros((pad,), jnp.int32)]
                           ).reshape(NW * CHUNKS, C)
    dstm = jnp.concatenate([dst, jnp.full((pad,), N_NODES, jnp.int32)]
                           ).reshape(NW * CHUNKS, C)
    x_pad = jnp.concatenate(
        [x, jnp.zeros((N_PAD - N_NODES, D), jnp.float32)], axis=0)
    bat_pad = jnp.concatenate(
        [batch.astype(jnp.int32),
         jnp.full((N_PAD - N_NODES,), NUM_GRAPHS, jnp.int32)])

    agg0, deg = _make_sc_agg(True)(x_pad, srcm, dstm)
    h1 = _layer_tc(agg0, deg, x_pad, W_l0, W_r0, b0)
    agg1 = _make_sc_agg(False)(h1, srcm, dstm)
    return _head_tc(agg1, deg, h1, W_l1, W_r1, b1, bat_pad,
                    mlp_W1, mlp_b1, mlp_W2, mlp_b2, cls_W, cls_b)


# speedup vs baseline: 1.0212x; 1.0212x over previous
import functools

import jax
import jax.numpy as jnp
from jax import lax
from jax.experimental import pallas as pl
from jax.experimental.pallas import tpu as pltpu
from jax.experimental.pallas import tpu_sc as plsc

N_NODES = 10000
N_EDGES = 320000
D = 128
NUM_GRAPHS = 64
D_OUT = 40

NC = 2
NS = 16
NW = NC * NS
C = 128
CHUNKS = 80
E_W = C * CHUNKS
E_PAD = NW * E_W
N_PAD = 10240
ROWS_W = N_PAD // NS
NBUF = 2
SEG = 8
CH0 = 144
CH1 = 16

_HIGH = jax.lax.Precision.HIGHEST


def _make_sc_agg(with_deg: bool):
    mesh = plsc.VectorSubcoreMesh(core_axis_name="c", subcore_axis_name="s",
                                  num_cores=NC)
    out_type = [jax.ShapeDtypeStruct((NC * N_PAD, D), jnp.float32)]
    if with_deg:
        out_type.append(jax.ShapeDtypeStruct((NW, N_PAD), jnp.float32))
    scratch = [
        pltpu.VMEM((2, SEG, C), jnp.int32),
        pltpu.VMEM((2, SEG, C), jnp.int32),
        pltpu.VMEM((NBUF, C, D), jnp.float32),
        pltpu.VMEM_SHARED((N_PAD, D), jnp.float32),
    ]
    scratch += [pltpu.SemaphoreType.DMA] * (2 * NBUF + 2)
    if with_deg:
        scratch.append(pltpu.VMEM((N_PAD,), jnp.float32))

    out_type_arg = tuple(out_type) if with_deg else out_type[0]

    @functools.partial(
        pl.kernel, mesh=mesh, out_type=out_type_arg,
        scratch_types=tuple(scratch),
        compiler_params=pltpu.CompilerParams(needs_layout_passes=False))
    def sc_agg(feat, srcm, dstm, *refs):
        if with_deg:
            (agg_out, deg_out, src_v, dst_v, rows_v, acc,
             *sems, deg_v) = refs
        else:
            agg_out, src_v, dst_v, rows_v, acc, *sems = refs
        gsem, ssem = sems[:NBUF], sems[NBUF:2 * NBUF]
        isem = sems[2 * NBUF:2 * NBUF + 2]
        ci = lax.axis_index("c")
        si = lax.axis_index("s")
        w = ci * NS + si

        zeros16 = jnp.zeros((16,), jnp.float32)

        def _zrow(i, carry):
            for k in range(D // 16):
                rows_v[0, i, pl.ds(k * 16, 16)] = zeros16
            return carry
        lax.fori_loop(0, C, _zrow, 0)

        if with_deg:
            def _zdeg(i, carry):
                deg_v[pl.ds(i * 16, 16)] = zeros16
                return carry
            lax.fori_loop(0, N_PAD // 16, _zdeg, 0)

        base = si * ROWS_W
        for t in range(ROWS_W // C):
            pltpu.sync_copy(rows_v.at[0], acc.at[pl.ds(base + t * C, C)])
        plsc.subcore_barrier()

        ones16 = jnp.ones((16,), jnp.float32)

        chunk_base = jnp.where(ci == 0, si * CH0, NS * CH0 + si * CH1)
        nseg_w = jnp.where(ci == 0, CH0 // SEG, CH1 // SEG)

        def _load_idx(s, p):
            seg_base = chunk_base + s * SEG
            pltpu.async_copy(srcm.at[pl.ds(seg_base, SEG)], src_v.at[p],
                             isem[0])
            pltpu.async_copy(dstm.at[pl.ds(seg_base, SEG)], dst_v.at[p],
                             isem[1])

        def _wait_idx(s, p):
            seg_base = chunk_base + s * SEG
            pltpu.make_async_copy(srcm.at[pl.ds(seg_base, SEG)], src_v.at[p],
                                  isem[0]).wait()
            pltpu.make_async_copy(dstm.at[pl.ds(seg_base, SEG)], dst_v.at[p],
                                  isem[1]).wait()

        @pl.when(0 < nseg_w)
        def _prime_idx():
            _load_idx(0, 0)

        def _segment(s, carry):
            p = lax.rem(s, 2)
            np_ = lax.rem(s + 1, 2)
            _wait_idx(s, p)

            @pl.when(s + 1 < nseg_w)
            def _prefetch_idx():
                _load_idx(s + 1, np_)

            pltpu.async_copy(feat.at[src_v.at[p, 0]], rows_v.at[0], gsem[0])
            for jj in range(SEG):
                b = jj % NBUF
                nb = (jj + 1) % NBUF
                pltpu.make_async_copy(
                    feat.at[src_v.at[p, jj]], rows_v.at[b], gsem[b]).wait()
                if jj + 1 < SEG:
                    if jj >= 1:
                        pltpu.make_async_copy(
                            rows_v.at[nb], acc.at[dst_v.at[p, jj]],
                            ssem[nb]).wait()
                    pltpu.async_copy(
                        feat.at[src_v.at[p, jj + 1]], rows_v.at[nb], gsem[nb])
                pltpu.async_copy(
                    rows_v.at[b], acc.at[dst_v.at[p, jj]], ssem[b], add=True)
                if with_deg:
                    for k in range(C // 16):
                        idx = dst_v[p, jj, pl.ds(k * 16, 16)]
                        plsc.addupdate_scatter(deg_v, [idx], ones16)
            for b in range(NBUF):
                pltpu.make_async_copy(
                    rows_v.at[b], acc.at[dst_v.at[p, SEG - 1]], ssem[b]).wait()
            return carry
        lax.fori_loop(0, nseg_w, _segment, 0)

        plsc.subcore_barrier()

        out_base = ci * N_PAD + base
        for t in range(ROWS_W // C):
            pltpu.sync_copy(acc.at[pl.ds(base + t * C, C)], rows_v.at[0])
            pltpu.sync_copy(rows_v.at[0], agg_out.at[pl.ds(out_base + t * C, C)])
        if with_deg:
            pltpu.sync_copy(deg_v, deg_out.at[w])

    return sc_agg


_make_sc_agg = functools.lru_cache(maxsize=None)(_make_sc_agg)


def _layer_body(agg_ref, deg_ref, x_ref, wl_ref, wr_ref, b_ref, h_ref):
    a = agg_ref[...]
    s = a[:N_PAD] + a[N_PAD:]
    deg = jnp.sum(deg_ref[...], axis=0)
    inv = 1.0 / jnp.maximum(deg, 1.0)
    m = s * inv[:, None]
    h = (jnp.dot(m, wl_ref[...], precision=_HIGH,
                 preferred_element_type=jnp.float32)
         + jnp.dot(x_ref[...], wr_ref[...], precision=_HIGH,
                   preferred_element_type=jnp.float32)
         + b_ref[...][None, :])
    h_ref[...] = jnp.maximum(h, 0.0)


_layer_tc = pl.pallas_call(
    _layer_body,
    out_shape=jax.ShapeDtypeStruct((N_PAD, D), jnp.float32),
)


def _head_body(agg_ref, deg_ref, h1_ref, wl_ref, wr_ref, b_ref,
               bat_ref, mw1_ref, mb1_ref, mw2_ref, mb2_ref,
               cw_ref, cb_ref, out_ref):
    a = agg_ref[...]
    s2 = a[:N_PAD] + a[N_PAD:]
    deg = jnp.sum(deg_ref[...], axis=0)
    inv = 1.0 / jnp.maximum(deg, 1.0)
    m = s2 * inv[:, None]
    h = jnp.maximum(
        jnp.dot(m, wl_ref[...], precision=_HIGH,
                preferred_element_type=jnp.float32)
        + jnp.dot(h1_ref[...], wr_ref[...], precision=_HIGH,
                  preferred_element_type=jnp.float32)
        + b_ref[...][None, :], 0.0)
    bat = bat_ref[...]
    ids = lax.broadcasted_iota(jnp.int32, (NUM_GRAPHS, N_PAD), 0)
    onehot = (ids == bat[None, :]).astype(jnp.float32)
    pooled = jnp.dot(onehot, h, precision=_HIGH,
                     preferred_element_type=jnp.float32)
    cnt = jnp.sum(onehot, axis=1)
    g = pooled / jnp.maximum(cnt, 1.0)[:, None]
    g = jnp.maximum(jnp.dot(g, mw1_ref[...], precision=_HIGH,
                            preferred_element_type=jnp.float32)
                    + mb1_ref[...][None, :], 0.0)
    g = jnp.maximum(jnp.dot(g, mw2_ref[...], precision=_HIGH,
                            preferred_element_type=jnp.float32)
                    + mb2_ref[...][None, :], 0.0)
    out_ref[...] = (jnp.dot(g, cw_ref[...], precision=_HIGH,
                            preferred_element_type=jnp.float32)
                    + cb_ref[...][None, :])


_head_tc = pl.pallas_call(
    _head_body,
    out_shape=jax.ShapeDtypeStruct((NUM_GRAPHS, D_OUT), jnp.float32),
)


def kernel(x, edge_index, batch, W_l0, W_r0, b0, W_l1, W_r1, b1,
           mlp_W1, mlp_b1, mlp_W2, mlp_b2, cls_W, cls_b):
    src = edge_index[0].astype(jnp.int32)
    dst = edge_index[1].astype(jnp.int32)
    pad = E_PAD - N_EDGES
    srcm = jnp.concatenate([src, jnp.zeros((pad,), jnp.int32)]
                           ).reshape(NW * CHUNKS, C)
    dstm = jnp.concatenate([dst, jnp.full((pad,), N_NODES, jnp.int32)]
                           ).reshape(NW * CHUNKS, C)
    x_pad = jnp.concatenate(
        [x, jnp.zeros((N_PAD - N_NODES, D), jnp.float32)], axis=0)
    bat_pad = jnp.concatenate(
        [batch.astype(jnp.int32),
         jnp.full((N_PAD - N_NODES,), NUM_GRAPHS, jnp.int32)])

    agg0, deg = _make_sc_agg(True)(x_pad, srcm, dstm)
    h1 = _layer_tc(agg0, deg, x_pad, W_l0, W_r0, b0)
    agg1 = _make_sc_agg(False)(h1, srcm, dstm)
    return _head_tc(agg1, deg, h1, W_l1, W_r1, b1, bat_pad,
                    mlp_W1, mlp_b1, mlp_W2, mlp_b2, cls_W, cls_b)
